# baseline (speedup 1.0000x reference)
import functools

import jax
import jax.numpy as jnp
from jax import lax
from jax.experimental import pallas as pl
from jax.experimental.pallas import tpu as pltpu
from jax.experimental.pallas import tpu_sc as plsc

_BATCH = 16384
_HIST = 200
_D = 16
_NW = 32
_B_PER_W = _BATCH // _NW
_NB = 8
_NITER = _B_PER_W // _NB
_SPLITS = ((0, 128), (128, 72))

_mesh = plsc.VectorSubcoreMesh(core_axis_name="c", subcore_axis_name="s")


@functools.partial(
    pl.kernel,
    mesh=_mesh,
    out_type=jax.ShapeDtypeStruct((_BATCH, _HIST, _D), jnp.float32),
    scratch_types=[
        pltpu.VMEM((2, _NB, _HIST), jnp.int32),
        pltpu.VMEM((2, _NB, _HIST, _D), jnp.float32),
        pltpu.SemaphoreType.DMA,
        pltpu.SemaphoreType.DMA,
        pltpu.SemaphoreType.DMA,
    ],
    compiler_params=pltpu.CompilerParams(use_tc_tiling_on_sc=False),
)
def _gather1(x_hbm, mu_hbm, out_mu,
             idx_v, mu_rows, sem_idx, sem_g, sem_w):
    cid = lax.axis_index("c")
    sid = lax.axis_index("s")
    wid = sid * 2 + cid
    b0 = wid * _B_PER_W

    def fire_gathers(slot):
        for i in range(_NB):
            for off, ln in _SPLITS:
                pltpu.async_copy(mu_hbm.at[idx_v.at[slot, i, pl.ds(off, ln)]],
                                 mu_rows.at[slot, i, pl.ds(off, ln)], sem_g)

    def drain_gathers(slot):
        for i in range(_NB):
            for off, ln in _SPLITS:
                pltpu.make_async_copy(
                    mu_hbm.at[idx_v.at[slot, i, pl.ds(off, ln)]],
                    mu_rows.at[slot, i, pl.ds(off, ln)], sem_g).wait()

    pltpu.sync_copy(x_hbm.at[pl.ds(b0, _NB)], idx_v.at[0])
    fire_gathers(0)

    def body(j, carry):
        s = j % 2
        ns = 1 - s
        b = b0 + j * _NB
        has_next = j + 1 < _NITER

        @pl.when(has_next)
        def _():
            pltpu.async_copy(x_hbm.at[pl.ds(b + _NB, _NB)],
                             idx_v.at[ns], sem_idx)

        drain_gathers(s)
        pltpu.async_copy(mu_rows.at[s], out_mu.at[pl.ds(b, _NB)], sem_w)

        @pl.when(has_next & (j > 0))
        def _():
            pltpu.make_async_copy(mu_rows.at[ns],
                                  out_mu.at[pl.ds(b - _NB, _NB)],
                                  sem_w).wait()

        @pl.when(has_next)
        def _():
            pltpu.make_async_copy(x_hbm.at[pl.ds(b + _NB, _NB)],
                                  idx_v.at[ns], sem_idx).wait()
            fire_gathers(ns)

        return carry

    lax.fori_loop(0, _NITER, body, 0)

    for jj in (_NITER - 2, _NITER - 1):
        s = jj % 2
        b = b0 + jj * _NB
        pltpu.make_async_copy(mu_rows.at[s],
                              out_mu.at[pl.ds(b, _NB)], sem_w).wait()


def kernel(x, embed_mu, embed_logvar):
    x32 = x.astype(jnp.int32)
    mu = _gather1(x32, embed_mu)
    logvar = _gather1(x32, embed_logvar)
    return (mu, mu, logvar)


# speedup vs baseline: 1.0117x; 1.0117x over previous
import functools

import jax
import jax.numpy as jnp
from jax import lax
from jax.experimental import pallas as pl
from jax.experimental.pallas import tpu as pltpu
from jax.experimental.pallas import tpu_sc as plsc

_BATCH = 16384
_HIST = 200
_D = 16
_NW = 32
_B_PER_W = _BATCH // _NW
_NB = 4
_NITER = _B_PER_W // _NB
_SPLITS = ((0, 128), (128, 72))

_mesh = plsc.VectorSubcoreMesh(core_axis_name="c", subcore_axis_name="s")


@functools.partial(
    pl.kernel,
    mesh=_mesh,
    out_type=jax.ShapeDtypeStruct((_BATCH, _HIST, _D), jnp.float32),
    scratch_types=[
        pltpu.VMEM((2, _NB, _HIST), jnp.int32),
        pltpu.VMEM((2, _NB, _HIST, _D), jnp.float32),
        pltpu.SemaphoreType.DMA,
        pltpu.SemaphoreType.DMA,
        pltpu.SemaphoreType.DMA,
    ],
    compiler_params=pltpu.CompilerParams(use_tc_tiling_on_sc=False),
)
def _gather1(x_hbm, mu_hbm, out_mu,
             idx_v, mu_rows, sem_idx, sem_g, sem_w):
    cid = lax.axis_index("c")
    sid = lax.axis_index("s")
    wid = sid * 2 + cid
    b0 = wid * _B_PER_W

    def fire_gathers(slot):
        for i in range(_NB):
            for off, ln in _SPLITS:
                pltpu.async_copy(mu_hbm.at[idx_v.at[slot, i, pl.ds(off, ln)]],
                                 mu_rows.at[slot, i, pl.ds(off, ln)], sem_g)

    def drain_gathers(slot):
        for i in range(_NB):
            for off, ln in _SPLITS:
                pltpu.make_async_copy(
                    mu_hbm.at[idx_v.at[slot, i, pl.ds(off, ln)]],
                    mu_rows.at[slot, i, pl.ds(off, ln)], sem_g).wait()

    pltpu.sync_copy(x_hbm.at[pl.ds(b0, _NB)], idx_v.at[0])
    fire_gathers(0)

    def body(j, carry):
        s = j % 2
        ns = 1 - s
        b = b0 + j * _NB
        has_next = j + 1 < _NITER

        @pl.when(has_next)
        def _():
            pltpu.async_copy(x_hbm.at[pl.ds(b + _NB, _NB)],
                             idx_v.at[ns], sem_idx)

        drain_gathers(s)
        pltpu.async_copy(mu_rows.at[s], out_mu.at[pl.ds(b, _NB)], sem_w)

        @pl.when(has_next & (j > 0))
        def _():
            pltpu.make_async_copy(mu_rows.at[ns],
                                  out_mu.at[pl.ds(b - _NB, _NB)],
                                  sem_w).wait()

        @pl.when(has_next)
        def _():
            pltpu.make_async_copy(x_hbm.at[pl.ds(b + _NB, _NB)],
                                  idx_v.at[ns], sem_idx).wait()
            fire_gathers(ns)

        return carry

    lax.fori_loop(0, _NITER, body, 0)

    for jj in (_NITER - 2, _NITER - 1):
        s = jj % 2
        b = b0 + jj * _NB
        pltpu.make_async_copy(mu_rows.at[s],
                              out_mu.at[pl.ds(b, _NB)], sem_w).wait()


def kernel(x, embed_mu, embed_logvar):
    x32 = x.astype(jnp.int32)
    mu = _gather1(x32, embed_mu)
    logvar = _gather1(x32, embed_logvar)
    return (mu, mu, logvar)
